# Initial kernel scaffold; baseline (speedup 1.0000x reference)
import jax
import jax.numpy as jnp
from jax.experimental import pallas as pl


def kernel(feat0, feat1, feat2, feat3, boxes):
    raise NotImplementedError("write your pallas kernel here")


# speedup vs baseline: 32.4437x; 32.4437x over previous
import math

import jax
import jax.numpy as jnp
from jax.experimental import pallas as pl
from jax.experimental.pallas import tpu as pltpu

_GRID = 192
_SCALE = 0.25
_IMG = 768.0
_OUT = 7
_SR = 2
_NS = _OUT * _SR
_K = 64
_POS_D = 720


def _make_body(h, w, off, bc):
    def body(bT_ref, f_ref, o_ref, rw_ref, cw_ref):
        pid = pl.program_id(0)

        @pl.when(pid == 0)
        def _build_weights():
            bT = bT_ref[...]
            x1 = bT[0:1, :] * _SCALE
            y1 = bT[1:2, :] * _SCALE
            x2 = bT[2:3, :] * _SCALE
            y2 = bT[3:4, :] * _SCALE
            roi_w = jnp.maximum(x2 - x1, 1.0)
            roi_h = jnp.maximum(y2 - y1, 1.0)
            g = (jax.lax.broadcasted_iota(jnp.int32, (_NS, _K), 0)
                 .astype(jnp.float32) + 0.5) / (2.0 * _OUT)
            xs = x1 + g * roi_w
            ys = y1 + g * roi_h

            def grid_tent(s):
                valid = ((s > -1.0) & (s < float(_GRID))).astype(jnp.float32)
                sc = jnp.clip(s, 0.0, float(_GRID - 1))
                i = jax.lax.broadcasted_iota(jnp.int32, (_GRID, _NS, _K),
                                             0).astype(jnp.float32)
                t = jnp.maximum(0.0, 1.0 - jnp.abs(sc[None] - i)) * valid[None]
                return jnp.sum(t, axis=1) * (1.0 / _NS)

            ay = grid_tent(ys)
            ax = grid_tent(xs)

            def up_tent(n):
                i = jax.lax.broadcasted_iota(jnp.int32, (n, _GRID),
                                             1).astype(jnp.float32)
                u = jnp.clip((i + 0.5) * (n / float(_GRID)) - 0.5, 0.0,
                             float(n - 1))
                r = jax.lax.broadcasted_iota(jnp.int32, (n, _GRID),
                                             0).astype(jnp.float32)
                return jnp.maximum(0.0, 1.0 - jnp.abs(u - r))

            rw_ref[...] = jnp.dot(up_tent(h), ay,
                                  preferred_element_type=jnp.float32)
            cw_ref[...] = jnp.dot(up_tent(w), ax,
                                  preferred_element_type=jnp.float32)

        x = f_ref[...].reshape(bc * h, w)
        b = jnp.dot(x, cw_ref[...], preferred_element_type=jnp.float32)
        pooled = jnp.sum(b.reshape(bc, h, _K) * rw_ref[...][None, :, :],
                         axis=1)

        c = off + pid * bc + jax.lax.broadcasted_iota(jnp.int32, (bc, 1), 0)
        blk = c // _POS_D
        j = c - blk * _POS_D
        expo = (2.0 * (j // 2).astype(jnp.float32)) / float(_POS_D)
        inv_t = jnp.exp(-math.log(10000.0) * expo)
        bT = bT_ref[...]
        nx1 = bT[0:1, :] / _IMG
        ny1 = bT[1:2, :] / _IMG
        nw = bT[2:3, :] / _IMG - nx1
        nh = bT[3:4, :] / _IMG - ny1
        cx = nx1 + nw * 0.5
        cy = ny1 + nh * 0.5
        v = jnp.where(blk == 0, cy,
                      jnp.where(blk == 1, cx,
                                jnp.where(blk == 2, nw, nh)))
        ang = v * (2.0 * math.pi) * inv_t
        pe = jnp.where(j % 2 == 0, jnp.sin(ang), jnp.cos(ang))

        o_ref[...] = pooled + pe

    return body


def _level_call(feat, boxes_t, off, bc):
    c, h, w = feat.shape
    body = _make_body(h, w, off, bc)
    return pl.pallas_call(
        body,
        out_shape=jax.ShapeDtypeStruct((c, _K), jnp.float32),
        grid=(c // bc,),
        in_specs=[
            pl.BlockSpec((4, _K), lambda i: (0, 0)),
            pl.BlockSpec((bc, h, w), lambda i: (i, 0, 0)),
        ],
        out_specs=pl.BlockSpec((bc, _K), lambda i: (i, 0)),
        scratch_shapes=[
            pltpu.VMEM((h, _K), jnp.float32),
            pltpu.VMEM((w, _K), jnp.float32),
        ],
        compiler_params=pltpu.CompilerParams(
            dimension_semantics=("arbitrary",)),
        name=f"roi_level_{h}",
    )(boxes_t, feat)


def kernel(feat0, feat1, feat2, feat3, boxes):
    boxes_t = jnp.transpose(boxes, (1, 0))
    outs = []
    off = 0
    for feat, bc in ((feat0, 16), (feat1, 32), (feat2, 64), (feat3, 128)):
        f = feat[0]
        outs.append(_level_call(f, boxes_t, off, bc))
        off += f.shape[0]
    full = jnp.concatenate(outs, axis=0)
    return jnp.transpose(full, (1, 0))[None]
